# baseline (speedup 1.0000x reference)
import jax
import jax.numpy as jnp
from jax.experimental import pallas as pl
from jax.experimental.pallas import tpu as pltpu

B = 4096
REPR_DIM = 2048
FEAT = 1024
STATE = 64
HID = 1024
GATE = 256
MHID = 256
E = 16
K = 2
ACT = 30

BT = 512


def _actor_kernel(obs_ref, sens_ref, trunk_W_ref, trunk_b_ref,
                  ln_g_ref, ln_b_ref, p1_Wa_ref, p1_Wb_ref, p1_b_ref,
                  gW1_ref, gb1_ref, gW2_ref, gb2_ref,
                  W1_ref, b1_ref, W2cat_ref, b2_ref, p2_W_ref, p2_b_ref,
                  mu_ref, aux_ref, imp_ref, load_ref):
    i = pl.program_id(0)
    t = jnp.dot(obs_ref[...], trunk_W_ref[...],
                preferred_element_type=jnp.float32) + trunk_b_ref[...]
    mu = jnp.mean(t, axis=-1, keepdims=True)
    var = jnp.mean((t - mu) ** 2, axis=-1, keepdims=True)
    h = (t - mu) * jax.lax.rsqrt(var + 1e-5) * ln_g_ref[...] + ln_b_ref[...]
    h = jnp.tanh(h)
    x = jnp.dot(h, p1_Wa_ref[...], preferred_element_type=jnp.float32)
    x = x + jnp.dot(sens_ref[...], p1_Wb_ref[...],
                    preferred_element_type=jnp.float32)
    x = jax.nn.relu(x + p1_b_ref[...])

    xb = x.astype(jnp.bfloat16)
    h1cat = jnp.concatenate(
        [jax.nn.relu(jnp.dot(xb, W1_ref[e], preferred_element_type=jnp.float32)
                     + b1_ref[e]) for e in range(E)], axis=-1)

    gh = jax.nn.relu(jnp.dot(x, gW1_ref[...],
                             preferred_element_type=jnp.float32) + gb1_ref[...])
    logits = jnp.dot(gh, gW2_ref[...],
                     preferred_element_type=jnp.float32) + gb2_ref[...]
    m = jnp.max(logits, axis=-1, keepdims=True)
    ex = jnp.exp(logits - m)
    probs = ex / jnp.sum(ex, axis=-1, keepdims=True)

    lane = jax.lax.broadcasted_iota(jnp.int32, probs.shape, 1)
    v1 = jnp.max(probs, axis=-1, keepdims=True)
    i1 = jnp.min(jnp.where(probs >= v1, lane, E), axis=-1, keepdims=True)
    mask1 = lane == i1
    pm = jnp.where(mask1, -1.0, probs)
    v2 = jnp.max(pm, axis=-1, keepdims=True)
    i2 = jnp.min(jnp.where(pm >= v2, lane, E), axis=-1, keepdims=True)
    mask2 = lane == i2
    den = v1 + v2 + 1e-9
    comb = jnp.where(mask1, v1 / den, 0.0) + jnp.where(mask2, v2 / den, 0.0)

    imp_part = jnp.sum(probs, axis=0, keepdims=True)
    load_part = jnp.sum(jnp.where(mask1, 1.0, 0.0) +
                        jnp.where(mask2, 1.0, 0.0), axis=0, keepdims=True)

    @pl.when(i == 0)
    def _init():
        imp_ref[...] = imp_part
        load_ref[...] = load_part

    @pl.when(i != 0)
    def _acc():
        imp_ref[...] += imp_part
        load_ref[...] += load_part

    rep_e = jax.lax.broadcasted_iota(jnp.int32, (E, E * MHID), 0)
    rep_c = jax.lax.broadcasted_iota(jnp.int32, (E, E * MHID), 1)
    R = jnp.where(rep_c // MHID == rep_e, 1.0, 0.0)
    comb_exp = jnp.dot(comb, R, preferred_element_type=jnp.float32)
    H1s = (h1cat * comb_exp).astype(jnp.bfloat16)
    y = jnp.dot(H1s, W2cat_ref[...], preferred_element_type=jnp.float32)
    y = y + jnp.dot(comb, b2_ref[...], preferred_element_type=jnp.float32)
    mu_ref[...] = jnp.tanh(
        jnp.dot(jax.nn.relu(y), p2_W_ref[...],
                preferred_element_type=jnp.float32) + p2_b_ref[...])

    @pl.when(i == pl.num_programs(0) - 1)
    def _finish():
        aux_ref[...] = E * jnp.sum((imp_ref[...] / B) * (load_ref[...] / (B * K)),
                                   axis=(0, 1), keepdims=True)


@jax.jit
def _run(obs, obs_sensor, trunk_W, trunk_b, ln_g, ln_b, p1_W, p1_b,
         gate_W1, gate_b1, gate_W2, gate_b2,
         exp_W1, exp_b1, exp_W2, exp_b2, p2_W, p2_b):
    trunk_b2 = trunk_b.reshape(1, FEAT)
    ln_g2 = ln_g.reshape(1, FEAT)
    ln_b2 = ln_b.reshape(1, FEAT)
    p1_Wa = p1_W[:FEAT]
    p1_Wb = p1_W[FEAT:]
    p1_b2 = p1_b.reshape(1, HID)
    gb1 = gate_b1.reshape(1, GATE)
    gb2 = gate_b2.reshape(1, E)
    W1b = exp_W1.astype(jnp.bfloat16)
    b1r = exp_b1.reshape(E, 1, MHID)
    W2cat = exp_W2.reshape(E * MHID, HID).astype(jnp.bfloat16)
    p2_b2 = p2_b.reshape(1, ACT)

    n = B // BT
    mu, aux, _, _ = pl.pallas_call(
        _actor_kernel,
        grid=(n,),
        in_specs=[
            pl.BlockSpec((BT, REPR_DIM), lambda i: (i, 0)),
            pl.BlockSpec((BT, STATE), lambda i: (i, 0)),
            pl.BlockSpec((REPR_DIM, FEAT), lambda i: (0, 0)),
            pl.BlockSpec((1, FEAT), lambda i: (0, 0)),
            pl.BlockSpec((1, FEAT), lambda i: (0, 0)),
            pl.BlockSpec((1, FEAT), lambda i: (0, 0)),
            pl.BlockSpec((FEAT, HID), lambda i: (0, 0)),
            pl.BlockSpec((STATE, HID), lambda i: (0, 0)),
            pl.BlockSpec((1, HID), lambda i: (0, 0)),
            pl.BlockSpec((HID, GATE), lambda i: (0, 0)),
            pl.BlockSpec((1, GATE), lambda i: (0, 0)),
            pl.BlockSpec((GATE, E), lambda i: (0, 0)),
            pl.BlockSpec((1, E), lambda i: (0, 0)),
            pl.BlockSpec((E, HID, MHID), lambda i: (0, 0, 0)),
            pl.BlockSpec((E, 1, MHID), lambda i: (0, 0, 0)),
            pl.BlockSpec((E * MHID, HID), lambda i: (0, 0)),
            pl.BlockSpec((E, HID), lambda i: (0, 0)),
            pl.BlockSpec((HID, ACT), lambda i: (0, 0)),
            pl.BlockSpec((1, ACT), lambda i: (0, 0)),
        ],
        out_specs=[
            pl.BlockSpec((BT, ACT), lambda i: (i, 0)),
            pl.BlockSpec((1, 1), lambda i: (0, 0)),
            pl.BlockSpec((1, E), lambda i: (0, 0)),
            pl.BlockSpec((1, E), lambda i: (0, 0)),
        ],
        out_shape=[
            jax.ShapeDtypeStruct((B, ACT), jnp.float32),
            jax.ShapeDtypeStruct((1, 1), jnp.float32),
            jax.ShapeDtypeStruct((1, E), jnp.float32),
            jax.ShapeDtypeStruct((1, E), jnp.float32),
        ],
    )(obs, obs_sensor, trunk_W, trunk_b2, ln_g2, ln_b2,
      p1_Wa, p1_Wb, p1_b2, gate_W1, gb1, gate_W2, gb2,
      W1b, b1r, W2cat, exp_b2, p2_W, p2_b2)

    return mu, aux[0, 0]


def kernel(obs, obs_sensor, std, trunk_W, trunk_b, ln_g, ln_b, p1_W, p1_b,
           gate_W1, gate_b1, gate_W2, gate_b2,
           exp_W1, exp_b1, exp_W2, exp_b2, p2_W, p2_b):
    mu, aux = _run(obs, obs_sensor, trunk_W, trunk_b, ln_g, ln_b, p1_W, p1_b,
                   gate_W1, gate_b1, gate_W2, gate_b2,
                   exp_W1, exp_b1, exp_W2, exp_b2, p2_W, p2_b)
    std_arr = jnp.ones_like(mu) * std
    return (mu, std_arr, aux)


# speedup vs baseline: 1.0517x; 1.0517x over previous
import jax
import jax.numpy as jnp
from jax.experimental import pallas as pl
from jax.experimental.pallas import tpu as pltpu

B = 4096
REPR_DIM = 2048
FEAT = 1024
STATE = 64
HID = 1024
GATE = 256
MHID = 256
E = 16
K = 2
ACT = 30

BT = 512


def _actor_kernel(obs_ref, sens_ref, trunk_W_ref, trunk_b_ref,
                  ln_g_ref, ln_b_ref, p1_Wa_ref, p1_Wb_ref, p1_b_ref,
                  gW1_ref, gb1_ref, gW2_ref, gb2_ref,
                  W1_ref, b1_ref, W2cat_ref, b2_ref, p2_W_ref, p2_b_ref,
                  mu_ref, aux_ref, imp_ref, load_ref):
    i = pl.program_id(0)
    t = jnp.dot(obs_ref[...], trunk_W_ref[...],
                preferred_element_type=jnp.float32) + trunk_b_ref[...]
    mu = jnp.mean(t, axis=-1, keepdims=True)
    var = jnp.mean((t - mu) ** 2, axis=-1, keepdims=True)
    h = (t - mu) * jax.lax.rsqrt(var + 1e-5) * ln_g_ref[...] + ln_b_ref[...]
    h = jnp.tanh(h)
    x = jnp.dot(h, p1_Wa_ref[...], preferred_element_type=jnp.float32)
    x = x + jnp.dot(sens_ref[...], p1_Wb_ref[...],
                    preferred_element_type=jnp.float32)
    x = jax.nn.relu(x + p1_b_ref[...])

    xb = x.astype(jnp.bfloat16)
    h1s = [jax.nn.relu(jnp.dot(xb, W1_ref[e],
                               preferred_element_type=jnp.float32)
                       + b1_ref[e]).astype(jnp.bfloat16)
           for e in range(E)]

    gh = jax.nn.relu(jnp.dot(x, gW1_ref[...],
                             preferred_element_type=jnp.float32) + gb1_ref[...])
    logits = jnp.dot(gh, gW2_ref[...],
                     preferred_element_type=jnp.float32) + gb2_ref[...]
    m = jnp.max(logits, axis=-1, keepdims=True)
    ex = jnp.exp(logits - m)
    probs = ex / jnp.sum(ex, axis=-1, keepdims=True)

    lane = jax.lax.broadcasted_iota(jnp.int32, probs.shape, 1)
    v1 = jnp.max(probs, axis=-1, keepdims=True)
    i1 = jnp.min(jnp.where(probs >= v1, lane, E), axis=-1, keepdims=True)
    mask1 = lane == i1
    pm = jnp.where(mask1, -1.0, probs)
    v2 = jnp.max(pm, axis=-1, keepdims=True)
    i2 = jnp.min(jnp.where(pm >= v2, lane, E), axis=-1, keepdims=True)
    mask2 = lane == i2
    den = v1 + v2 + 1e-9
    comb = jnp.where(mask1, v1 / den, 0.0) + jnp.where(mask2, v2 / den, 0.0)

    imp_part = jnp.sum(probs, axis=0, keepdims=True)
    load_part = jnp.sum(jnp.where(mask1, 1.0, 0.0) +
                        jnp.where(mask2, 1.0, 0.0), axis=0, keepdims=True)

    @pl.when(i == 0)
    def _init():
        imp_ref[...] = imp_part
        load_ref[...] = load_part

    @pl.when(i != 0)
    def _acc():
        imp_ref[...] += imp_part
        load_ref[...] += load_part

    comb_bf = comb.astype(jnp.bfloat16)
    H1s = jnp.concatenate(
        [h1s[e] * comb_bf[:, e:e + 1] for e in range(E)], axis=-1)
    y = jnp.dot(H1s, W2cat_ref[...], preferred_element_type=jnp.float32)
    y = y + jnp.dot(comb, b2_ref[...], preferred_element_type=jnp.float32)
    mu_ref[...] = jnp.tanh(
        jnp.dot(jax.nn.relu(y), p2_W_ref[...],
                preferred_element_type=jnp.float32) + p2_b_ref[...])

    @pl.when(i == pl.num_programs(0) - 1)
    def _finish():
        aux_ref[...] = E * jnp.sum((imp_ref[...] / B) * (load_ref[...] / (B * K)),
                                   axis=(0, 1), keepdims=True)


@jax.jit
def _run(obs, obs_sensor, trunk_W, trunk_b, ln_g, ln_b, p1_W, p1_b,
         gate_W1, gate_b1, gate_W2, gate_b2,
         exp_W1, exp_b1, exp_W2, exp_b2, p2_W, p2_b):
    trunk_b2 = trunk_b.reshape(1, FEAT)
    ln_g2 = ln_g.reshape(1, FEAT)
    ln_b2 = ln_b.reshape(1, FEAT)
    p1_Wa = p1_W[:FEAT]
    p1_Wb = p1_W[FEAT:]
    p1_b2 = p1_b.reshape(1, HID)
    gb1 = gate_b1.reshape(1, GATE)
    gb2 = gate_b2.reshape(1, E)
    W1b = exp_W1.astype(jnp.bfloat16)
    b1r = exp_b1.reshape(E, 1, MHID)
    W2cat = exp_W2.reshape(E * MHID, HID).astype(jnp.bfloat16)
    p2_b2 = p2_b.reshape(1, ACT)

    n = B // BT
    mu, aux, _, _ = pl.pallas_call(
        _actor_kernel,
        grid=(n,),
        in_specs=[
            pl.BlockSpec((BT, REPR_DIM), lambda i: (i, 0)),
            pl.BlockSpec((BT, STATE), lambda i: (i, 0)),
            pl.BlockSpec((REPR_DIM, FEAT), lambda i: (0, 0)),
            pl.BlockSpec((1, FEAT), lambda i: (0, 0)),
            pl.BlockSpec((1, FEAT), lambda i: (0, 0)),
            pl.BlockSpec((1, FEAT), lambda i: (0, 0)),
            pl.BlockSpec((FEAT, HID), lambda i: (0, 0)),
            pl.BlockSpec((STATE, HID), lambda i: (0, 0)),
            pl.BlockSpec((1, HID), lambda i: (0, 0)),
            pl.BlockSpec((HID, GATE), lambda i: (0, 0)),
            pl.BlockSpec((1, GATE), lambda i: (0, 0)),
            pl.BlockSpec((GATE, E), lambda i: (0, 0)),
            pl.BlockSpec((1, E), lambda i: (0, 0)),
            pl.BlockSpec((E, HID, MHID), lambda i: (0, 0, 0)),
            pl.BlockSpec((E, 1, MHID), lambda i: (0, 0, 0)),
            pl.BlockSpec((E * MHID, HID), lambda i: (0, 0)),
            pl.BlockSpec((E, HID), lambda i: (0, 0)),
            pl.BlockSpec((HID, ACT), lambda i: (0, 0)),
            pl.BlockSpec((1, ACT), lambda i: (0, 0)),
        ],
        out_specs=[
            pl.BlockSpec((BT, ACT), lambda i: (i, 0)),
            pl.BlockSpec((1, 1), lambda i: (0, 0)),
            pl.BlockSpec((1, E), lambda i: (0, 0)),
            pl.BlockSpec((1, E), lambda i: (0, 0)),
        ],
        out_shape=[
            jax.ShapeDtypeStruct((B, ACT), jnp.float32),
            jax.ShapeDtypeStruct((1, 1), jnp.float32),
            jax.ShapeDtypeStruct((1, E), jnp.float32),
            jax.ShapeDtypeStruct((1, E), jnp.float32),
        ],
    )(obs, obs_sensor, trunk_W, trunk_b2, ln_g2, ln_b2,
      p1_Wa, p1_Wb, p1_b2, gate_W1, gb1, gate_W2, gb2,
      W1b, b1r, W2cat, exp_b2, p2_W, p2_b2)

    return mu, aux[0, 0]


def kernel(obs, obs_sensor, std, trunk_W, trunk_b, ln_g, ln_b, p1_W, p1_b,
           gate_W1, gate_b1, gate_W2, gate_b2,
           exp_W1, exp_b1, exp_W2, exp_b2, p2_W, p2_b):
    mu, aux = _run(obs, obs_sensor, trunk_W, trunk_b, ln_g, ln_b, p1_W, p1_b,
                   gate_W1, gate_b1, gate_W2, gate_b2,
                   exp_W1, exp_b1, exp_W2, exp_b2, p2_W, p2_b)
    std_arr = jnp.ones_like(mu) * std
    return (mu, std_arr, aux)
